# baseline (speedup 1.0000x reference)
import functools

import jax
import jax.numpy as jnp
from jax import lax
from jax.experimental import pallas as pl
from jax.experimental.pallas import tpu as pltpu
from jax.experimental.pallas import tpu_sc as plsc

N = 10000
NPAD = 10240
D = 128
E = 320000
NC = 2
NS = 16
EPT = E // (NC * NS)
CH = 40
NCHUNK = EPT // CH
RPT = NPAD // NS
ZR = 16

_MESH = plsc.VectorSubcoreMesh(
    core_axis_name="c", subcore_axis_name="s", num_cores=NC, num_subcores=NS
)


@functools.partial(
    pl.kernel,
    out_type=jax.ShapeDtypeStruct((NC * NS * NPAD,), jnp.float32),
    mesh=_MESH,
    scratch_types=[
        pltpu.VMEM((EPT,), jnp.int32),
        [pltpu.VMEM((NPAD,), jnp.float32)] * 4,
    ],
)
def _deg_kernel(dst_hbm, out_hbm, idx_v, hists):
    c = lax.axis_index("c")
    s = lax.axis_index("s")
    wid = c * NS + s

    def zfill(i, carry):
        for h in hists:
            h[pl.ds(i * 16, 16)] = jnp.zeros((16,), jnp.float32)
        return carry

    lax.fori_loop(0, NPAD // 16, zfill, 0)
    pltpu.sync_copy(dst_hbm.at[pl.ds(wid * EPT, EPT)], idx_v)
    lanes = lax.iota(jnp.int32, 16)

    def body(j, carry):
        dvec = idx_v[pl.ds(j * 16, 16)]
        for k in range(16):
            d = dvec[k]
            row = (d // 16) * 16
            oh = jnp.where(lanes == d % 16, 1.0, 0.0).astype(jnp.float32)
            h = hists[k % 4]
            h[pl.ds(row, 16)] = h[pl.ds(row, 16)] + oh
        return carry

    lax.fori_loop(0, EPT // 16, body, 0)

    def rsum(i, carry):
        sl = pl.ds(i * 16, 16)
        hists[0][sl] = (hists[0][sl] + hists[1][sl]) + (
            hists[2][sl] + hists[3][sl]
        )
        return carry

    lax.fori_loop(0, NPAD // 16, rsum, 0)
    pltpu.sync_copy(hists[0], out_hbm.at[pl.ds(wid * NPAD, NPAD)])


NBUF = 5
NOUT = NCHUNK // NBUF


@functools.partial(
    pl.kernel,
    out_type=jax.ShapeDtypeStruct((NC * NPAD, D), jnp.float32),
    mesh=_MESH,
    scratch_types=[
        pltpu.VMEM((EPT,), jnp.int32),
        pltpu.VMEM((EPT,), jnp.int32),
        pltpu.VMEM((NBUF, CH, D), jnp.float32),
        pltpu.VMEM((ZR, D), jnp.float32),
        pltpu.VMEM_SHARED((NPAD, D), jnp.float32),
        [pltpu.SemaphoreType.DMA] * NBUF,
        [pltpu.SemaphoreType.DMA] * NBUF,
        pltpu.SemaphoreType.DMA,
    ],
)
def _agg_kernel(g_hbm, src_hbm, dst_hbm, out_hbm, srcb, dstb, rows, zero_v,
                acc_sh, gsems, ssems, fsem):
    c = lax.axis_index("c")
    s = lax.axis_index("s")
    wid = c * NS + s
    for r in range(ZR):
        for k in range(D // 16):
            zero_v[r, pl.ds(k * 16, 16)] = jnp.zeros((16,), jnp.float32)

    nz = RPT // ZR
    for g in range(2):
        zds = []
        for i in range(nz // 2):
            r0 = s * RPT + (g * (nz // 2) + i) * ZR
            zds.append(pltpu.async_copy(zero_v, acc_sh.at[pl.ds(r0, ZR)],
                                        fsem))
        for d in zds:
            d.wait()
    plsc.subcore_barrier()

    ebase = wid * EPT
    pltpu.sync_copy(src_hbm.at[pl.ds(ebase, EPT)], srcb)
    pltpu.sync_copy(dst_hbm.at[pl.ds(ebase, EPT)], dstb)

    def body(t, carry):
        e0 = t * (NBUF * CH)
        gds = []
        for b in range(NBUF):
            @pl.when(t > 0)
            def _(b=b):
                pltpu.make_async_copy(
                    rows.at[b], acc_sh.at[pl.ds(0, CH)], ssems[b]
                ).wait()

            gds.append(pltpu.async_copy(
                g_hbm.at[srcb.at[pl.ds(e0 + b * CH, CH)]],
                rows.at[b], gsems[b]))
        for b in range(NBUF):
            gds[b].wait()
            pltpu.async_copy(
                rows.at[b],
                acc_sh.at[dstb.at[pl.ds(e0 + b * CH, CH)]],
                ssems[b], add=True)
        return carry

    lax.fori_loop(0, NOUT, body, 0)
    for b in range(NBUF):
        pltpu.make_async_copy(
            rows.at[b], acc_sh.at[pl.ds(0, CH)], ssems[b]
        ).wait()
    plsc.subcore_barrier()

    nf = RPT // CH
    def fbody(t, carry):
        fds = []
        for b in range(4):
            r0 = s * RPT + (t * 4 + b) * CH
            pltpu.sync_copy(acc_sh.at[pl.ds(r0, CH)], rows.at[b])
            fds.append(pltpu.async_copy(
                rows.at[b], out_hbm.at[pl.ds(c * NPAD + r0, CH)], gsems[b]))
        for d in fds:
            d.wait()
        return carry

    lax.fori_loop(0, nf // 4, fbody, 0)


_BLK = 512
_GRID = NPAD // _BLK


def _tca_body(x_ref, w_ref, dp_ref, g_ref, dinv_ref):
    deg = jnp.sum(dp_ref[...], axis=0) + 1.0
    dinv = lax.rsqrt(deg)[:, None]
    g_ref[...] = jnp.dot(
        x_ref[...], w_ref[...], preferred_element_type=jnp.float32
    ) * dinv
    dinv_ref[...] = dinv


def _tca(xpad, w1, degp):
    return pl.pallas_call(
        _tca_body,
        grid=(_GRID,),
        in_specs=[
            pl.BlockSpec((_BLK, D), lambda i: (i, 0)),
            pl.BlockSpec((D, D), lambda i: (0, 0)),
            pl.BlockSpec((NC * NS, _BLK), lambda i: (0, i)),
        ],
        out_specs=[
            pl.BlockSpec((_BLK, D), lambda i: (i, 0)),
            pl.BlockSpec((_BLK, 1), lambda i: (i, 0)),
        ],
        out_shape=[
            jax.ShapeDtypeStruct((NPAD, D), jnp.float32),
            jax.ShapeDtypeStruct((NPAD, 1), jnp.float32),
        ],
    )(xpad, w1, degp)


def _tcb_body(p0_ref, p1_ref, g1_ref, dinv_ref, b1_ref, w2_ref, g2_ref):
    ssum = p0_ref[...] + p1_ref[...] + g1_ref[...]
    z = jnp.maximum(ssum * dinv_ref[...] + b1_ref[...], 0.0)
    g2_ref[...] = jnp.dot(
        z, w2_ref[...], preferred_element_type=jnp.float32
    ) * dinv_ref[...]


def _tcb(p1, g1, dinv, b1, w2):
    return pl.pallas_call(
        _tcb_body,
        grid=(_GRID,),
        in_specs=[
            pl.BlockSpec((_BLK, D), lambda i: (i, 0)),
            pl.BlockSpec((_BLK, D), lambda i: (i + _GRID, 0)),
            pl.BlockSpec((_BLK, D), lambda i: (i, 0)),
            pl.BlockSpec((_BLK, 1), lambda i: (i, 0)),
            pl.BlockSpec((1, D), lambda i: (0, 0)),
            pl.BlockSpec((D, D), lambda i: (0, 0)),
        ],
        out_specs=pl.BlockSpec((_BLK, D), lambda i: (i, 0)),
        out_shape=jax.ShapeDtypeStruct((NPAD, D), jnp.float32),
    )(p1, p1, g1, dinv, b1, w2)


def _tcc_body(p0_ref, p1_ref, g2_ref, dinv_ref, b2_ref, out_ref):
    ssum = p0_ref[...] + p1_ref[...] + g2_ref[...]
    out_ref[...] = ssum * dinv_ref[...] + b2_ref[...]


def _tcc(p2, g2, dinv, b2):
    return pl.pallas_call(
        _tcc_body,
        grid=(_GRID,),
        in_specs=[
            pl.BlockSpec((_BLK, D), lambda i: (i, 0)),
            pl.BlockSpec((_BLK, D), lambda i: (i + _GRID, 0)),
            pl.BlockSpec((_BLK, D), lambda i: (i, 0)),
            pl.BlockSpec((_BLK, 1), lambda i: (i, 0)),
            pl.BlockSpec((1, D), lambda i: (0, 0)),
        ],
        out_specs=pl.BlockSpec((_BLK, D), lambda i: (i, 0)),
        out_shape=jax.ShapeDtypeStruct((NPAD, D), jnp.float32),
    )(p2, p2, g2, dinv, b2)


def kernel(x, edge_index, W1, b1, W2, b2):
    src = edge_index[0].astype(jnp.int32)
    dst = edge_index[1].astype(jnp.int32)
    xpad = jnp.pad(x, ((0, NPAD - N), (0, 0)))
    b1r = b1.reshape(1, D)
    b2r = b2.reshape(1, D)

    degp = _deg_kernel(dst).reshape(NC * NS, NPAD)
    g1, dinv = _tca(xpad, W1, degp)
    p1 = _agg_kernel(g1, src, dst)
    g2 = _tcb(p1, g1, dinv, b1r, W2)
    p2 = _agg_kernel(g2, src, dst)
    outp = _tcc(p2, g2, dinv, b2r)
    return outp[:N]


# speedup vs baseline: 1.0150x; 1.0150x over previous
import functools

import jax
import jax.numpy as jnp
from jax import lax
from jax.experimental import pallas as pl
from jax.experimental.pallas import tpu as pltpu
from jax.experimental.pallas import tpu_sc as plsc

N = 10000
NPAD = 10240
D = 128
E = 320000
NC = 2
NS = 16
EPT = E // (NC * NS)
CH = 40
NCHUNK = EPT // CH
RPT = NPAD // NS
ZR = 16

_MESH = plsc.VectorSubcoreMesh(
    core_axis_name="c", subcore_axis_name="s", num_cores=NC, num_subcores=NS
)


@functools.partial(
    pl.kernel,
    out_type=jax.ShapeDtypeStruct((NC * NS * NPAD,), jnp.float32),
    mesh=_MESH,
    scratch_types=[
        pltpu.VMEM((EPT,), jnp.int32),
        [pltpu.VMEM((NPAD,), jnp.float32)] * 4,
    ],
)
def _deg_kernel(dst_hbm, out_hbm, idx_v, hists):
    c = lax.axis_index("c")
    s = lax.axis_index("s")
    wid = c * NS + s

    def zfill(i, carry):
        for h in hists:
            h[pl.ds(i * 16, 16)] = jnp.zeros((16,), jnp.float32)
        return carry

    lax.fori_loop(0, NPAD // 16, zfill, 0)
    pltpu.sync_copy(dst_hbm.at[pl.ds(wid * EPT, EPT)], idx_v)
    lanes = lax.iota(jnp.int32, 16)

    def body(j, carry):
        dvec = idx_v[pl.ds(j * 16, 16)]
        for k in range(16):
            d = dvec[k]
            row = (d // 16) * 16
            oh = jnp.where(lanes == d % 16, 1.0, 0.0).astype(jnp.float32)
            h = hists[k % 4]
            h[pl.ds(row, 16)] = h[pl.ds(row, 16)] + oh
        return carry

    lax.fori_loop(0, EPT // 16, body, 0)

    def rsum(i, carry):
        sl = pl.ds(i * 16, 16)
        hists[0][sl] = (hists[0][sl] + hists[1][sl]) + (
            hists[2][sl] + hists[3][sl]
        )
        return carry

    lax.fori_loop(0, NPAD // 16, rsum, 0)
    pltpu.sync_copy(hists[0], out_hbm.at[pl.ds(wid * NPAD, NPAD)])


NBUF = 5
NOUT = NCHUNK // NBUF


@functools.partial(
    pl.kernel,
    out_type=jax.ShapeDtypeStruct((NC * NPAD, D), jnp.float32),
    mesh=_MESH,
    scratch_types=[
        pltpu.VMEM((EPT,), jnp.int32),
        pltpu.VMEM((EPT,), jnp.int32),
        pltpu.VMEM((NBUF, CH, D), jnp.float32),
        pltpu.VMEM((ZR, D), jnp.float32),
        pltpu.VMEM_SHARED((NPAD, D), jnp.float32),
        [pltpu.SemaphoreType.DMA] * NBUF,
        [pltpu.SemaphoreType.DMA] * NBUF,
        pltpu.SemaphoreType.DMA,
    ],
)
def _agg_kernel(g_hbm, src_hbm, dst_hbm, out_hbm, srcb, dstb, rows, zero_v,
                acc_sh, gsems, ssems, fsem):
    c = lax.axis_index("c")
    s = lax.axis_index("s")
    wid = c * NS + s
    for r in range(ZR):
        for k in range(D // 16):
            zero_v[r, pl.ds(k * 16, 16)] = jnp.zeros((16,), jnp.float32)

    ebase = wid * EPT
    pd0 = pltpu.async_copy(src_hbm.at[pl.ds(ebase, EPT)], srcb, gsems[0])
    pd1 = pltpu.async_copy(dst_hbm.at[pl.ds(ebase, EPT)], dstb, gsems[1])

    nz = RPT // ZR
    for g in range(2):
        zds = []
        for i in range(nz // 2):
            r0 = s * RPT + (g * (nz // 2) + i) * ZR
            zds.append(pltpu.async_copy(zero_v, acc_sh.at[pl.ds(r0, ZR)],
                                        fsem))
        for d in zds:
            d.wait()
    pd0.wait()
    pd1.wait()
    plsc.subcore_barrier()

    def body(t, carry):
        e0 = t * (NBUF * CH)
        gds = []
        for b in range(NBUF):
            @pl.when(t > 0)
            def _(b=b):
                pltpu.make_async_copy(
                    rows.at[b], acc_sh.at[pl.ds(0, CH)], ssems[b]
                ).wait()

            gds.append(pltpu.async_copy(
                g_hbm.at[srcb.at[pl.ds(e0 + b * CH, CH)]],
                rows.at[b], gsems[b]))
        for b in range(NBUF):
            gds[b].wait()
            pltpu.async_copy(
                rows.at[b],
                acc_sh.at[dstb.at[pl.ds(e0 + b * CH, CH)]],
                ssems[b], add=True)
        return carry

    lax.fori_loop(0, NOUT, body, 0)
    for b in range(NBUF):
        pltpu.make_async_copy(
            rows.at[b], acc_sh.at[pl.ds(0, CH)], ssems[b]
        ).wait()
    plsc.subcore_barrier()

    nf = RPT // CH
    def fbody(t, carry):
        fds = []
        for b in range(4):
            r0 = s * RPT + (t * 4 + b) * CH
            pltpu.sync_copy(acc_sh.at[pl.ds(r0, CH)], rows.at[b])
            fds.append(pltpu.async_copy(
                rows.at[b], out_hbm.at[pl.ds(c * NPAD + r0, CH)], gsems[b]))
        for d in fds:
            d.wait()
        return carry

    lax.fori_loop(0, nf // 4, fbody, 0)


_BLK = 512
_GRID = NPAD // _BLK


def _tcmm_body(x_ref, w_ref, h_ref):
    h_ref[...] = jnp.dot(
        x_ref[...], w_ref[...], preferred_element_type=jnp.float32
    )


def _tcmm(xpad, w1):
    return pl.pallas_call(
        _tcmm_body,
        grid=(_GRID,),
        in_specs=[
            pl.BlockSpec((_BLK, D), lambda i: (i, 0)),
            pl.BlockSpec((D, D), lambda i: (0, 0)),
        ],
        out_specs=pl.BlockSpec((_BLK, D), lambda i: (i, 0)),
        out_shape=jax.ShapeDtypeStruct((NPAD, D), jnp.float32),
    )(xpad, w1)


def _tca_body(h_ref, dp_ref, g_ref, dinv_ref):
    deg = jnp.sum(dp_ref[...], axis=0) + 1.0
    dinv = lax.rsqrt(deg)[:, None]
    g_ref[...] = h_ref[...] * dinv
    dinv_ref[...] = dinv


def _tca(h1, degp):
    return pl.pallas_call(
        _tca_body,
        grid=(_GRID,),
        in_specs=[
            pl.BlockSpec((_BLK, D), lambda i: (i, 0)),
            pl.BlockSpec((NC * NS, _BLK), lambda i: (0, i)),
        ],
        out_specs=[
            pl.BlockSpec((_BLK, D), lambda i: (i, 0)),
            pl.BlockSpec((_BLK, 1), lambda i: (i, 0)),
        ],
        out_shape=[
            jax.ShapeDtypeStruct((NPAD, D), jnp.float32),
            jax.ShapeDtypeStruct((NPAD, 1), jnp.float32),
        ],
    )(h1, degp)


def _tcb_body(p0_ref, p1_ref, g1_ref, dinv_ref, b1_ref, w2_ref, g2_ref):
    ssum = p0_ref[...] + p1_ref[...] + g1_ref[...]
    z = jnp.maximum(ssum * dinv_ref[...] + b1_ref[...], 0.0)
    g2_ref[...] = jnp.dot(
        z, w2_ref[...], preferred_element_type=jnp.float32
    ) * dinv_ref[...]


def _tcb(p1, g1, dinv, b1, w2):
    return pl.pallas_call(
        _tcb_body,
        grid=(_GRID,),
        in_specs=[
            pl.BlockSpec((_BLK, D), lambda i: (i, 0)),
            pl.BlockSpec((_BLK, D), lambda i: (i + _GRID, 0)),
            pl.BlockSpec((_BLK, D), lambda i: (i, 0)),
            pl.BlockSpec((_BLK, 1), lambda i: (i, 0)),
            pl.BlockSpec((1, D), lambda i: (0, 0)),
            pl.BlockSpec((D, D), lambda i: (0, 0)),
        ],
        out_specs=pl.BlockSpec((_BLK, D), lambda i: (i, 0)),
        out_shape=jax.ShapeDtypeStruct((NPAD, D), jnp.float32),
    )(p1, p1, g1, dinv, b1, w2)


def _tcc_body(p0_ref, p1_ref, g2_ref, dinv_ref, b2_ref, out_ref):
    ssum = p0_ref[...] + p1_ref[...] + g2_ref[...]
    out_ref[...] = ssum * dinv_ref[...] + b2_ref[...]


def _tcc(p2, g2, dinv, b2):
    return pl.pallas_call(
        _tcc_body,
        grid=(_GRID,),
        in_specs=[
            pl.BlockSpec((_BLK, D), lambda i: (i, 0)),
            pl.BlockSpec((_BLK, D), lambda i: (i + _GRID, 0)),
            pl.BlockSpec((_BLK, D), lambda i: (i, 0)),
            pl.BlockSpec((_BLK, 1), lambda i: (i, 0)),
            pl.BlockSpec((1, D), lambda i: (0, 0)),
        ],
        out_specs=pl.BlockSpec((_BLK, D), lambda i: (i, 0)),
        out_shape=jax.ShapeDtypeStruct((NPAD, D), jnp.float32),
    )(p2, p2, g2, dinv, b2)


def kernel(x, edge_index, W1, b1, W2, b2):
    src = edge_index[0].astype(jnp.int32)
    dst = edge_index[1].astype(jnp.int32)
    xpad = jnp.pad(x, ((0, NPAD - N), (0, 0)))
    b1r = b1.reshape(1, D)
    b2r = b2.reshape(1, D)

    h1 = _tcmm(xpad, W1)
    degp = _deg_kernel(dst).reshape(NC * NS, NPAD)
    g1, dinv = _tca(h1, degp)
    p1 = _agg_kernel(g1, src, dst)
    g2 = _tcb(p1, g1, dinv, b1r, W2)
    p2 = _agg_kernel(g2, src, dst)
    outp = _tcc(p2, g2, dinv, b2r)
    return outp[:N]


# speedup vs baseline: 1.0572x; 1.0416x over previous
import functools

import jax
import jax.numpy as jnp
from jax import lax
from jax.experimental import pallas as pl
from jax.experimental.pallas import tpu as pltpu
from jax.experimental.pallas import tpu_sc as plsc

N = 10000
NPAD = 10240
D = 128
E = 320000
NC = 2
NS = 16
EPT = E // (NC * NS)
CH = 40
NCHUNK = EPT // CH
RPT = NPAD // NS
ZR = 16

_MESH = plsc.VectorSubcoreMesh(
    core_axis_name="c", subcore_axis_name="s", num_cores=NC, num_subcores=NS
)


@functools.partial(
    pl.kernel,
    out_type=jax.ShapeDtypeStruct((NC * NS * NPAD,), jnp.float32),
    mesh=_MESH,
    scratch_types=[
        pltpu.VMEM((EPT,), jnp.int32),
        [pltpu.VMEM((NPAD,), jnp.float32)] * 4,
    ],
)
def _deg_kernel(dst_hbm, out_hbm, idx_v, hists):
    c = lax.axis_index("c")
    s = lax.axis_index("s")
    wid = c * NS + s

    def zfill(i, carry):
        for h in hists:
            h[pl.ds(i * 16, 16)] = jnp.zeros((16,), jnp.float32)
        return carry

    lax.fori_loop(0, NPAD // 16, zfill, 0)
    pltpu.sync_copy(dst_hbm.at[pl.ds(wid * EPT, EPT)], idx_v)
    lanes = lax.iota(jnp.int32, 16)

    def body(j, carry):
        dvec = idx_v[pl.ds(j * 16, 16)]
        for k in range(16):
            d = dvec[k]
            row = (d // 16) * 16
            oh = jnp.where(lanes == d % 16, 1.0, 0.0).astype(jnp.float32)
            h = hists[k % 4]
            h[pl.ds(row, 16)] = h[pl.ds(row, 16)] + oh
        return carry

    lax.fori_loop(0, EPT // 16, body, 0)

    def rsum(i, carry):
        sl = pl.ds(i * 16, 16)
        hists[0][sl] = (hists[0][sl] + hists[1][sl]) + (
            hists[2][sl] + hists[3][sl]
        )
        return carry

    lax.fori_loop(0, NPAD // 16, rsum, 0)
    pltpu.sync_copy(hists[0], out_hbm.at[pl.ds(wid * NPAD, NPAD)])


NBUF = 5
NOUT = NCHUNK // NBUF


@functools.partial(
    pl.kernel,
    out_type=jax.ShapeDtypeStruct((NC * NPAD, D), jnp.float32),
    mesh=_MESH,
    scratch_types=[
        pltpu.VMEM((EPT,), jnp.int32),
        pltpu.VMEM((EPT,), jnp.int32),
        pltpu.VMEM((NBUF, CH, D), jnp.float32),
        pltpu.VMEM((ZR, D), jnp.float32),
        pltpu.VMEM_SHARED((NPAD, D), jnp.float32),
        [pltpu.SemaphoreType.DMA] * NBUF,
        [pltpu.SemaphoreType.DMA] * NBUF,
        pltpu.SemaphoreType.DMA,
    ],
)
def _agg_kernel(g_hbm, src_hbm, dst_hbm, out_hbm, srcb, dstb, rows, zero_v,
                acc_sh, gsems, ssems, fsem):
    c = lax.axis_index("c")
    s = lax.axis_index("s")
    wid = c * NS + s
    for r in range(ZR):
        for k in range(D // 16):
            zero_v[r, pl.ds(k * 16, 16)] = jnp.zeros((16,), jnp.float32)

    ebase = wid * EPT
    pd0 = pltpu.async_copy(src_hbm.at[pl.ds(ebase, EPT)], srcb, gsems[0])
    pd1 = pltpu.async_copy(dst_hbm.at[pl.ds(ebase, EPT)], dstb, gsems[1])

    nz = RPT // ZR
    for g in range(2):
        zds = []
        for i in range(nz // 2):
            r0 = s * RPT + (g * (nz // 2) + i) * ZR
            zds.append(pltpu.async_copy(zero_v, acc_sh.at[pl.ds(r0, ZR)],
                                        fsem))
        for d in zds:
            d.wait()
    pd0.wait()
    pd1.wait()
    plsc.subcore_barrier()

    def body(t, carry):
        e0 = t * (NBUF * CH)
        gds = []
        for b in range(NBUF):
            @pl.when(t > 0)
            def _(b=b):
                pltpu.make_async_copy(
                    rows.at[b], acc_sh.at[pl.ds(0, CH)], ssems[b]
                ).wait()

            gds.append(pltpu.async_copy(
                g_hbm.at[srcb.at[pl.ds(e0 + b * CH, CH)]],
                rows.at[b], gsems[b]))
        for b in range(NBUF):
            gds[b].wait()
            pltpu.async_copy(
                rows.at[b],
                acc_sh.at[dstb.at[pl.ds(e0 + b * CH, CH)]],
                ssems[b], add=True)
        return carry

    lax.fori_loop(0, NOUT, body, 0)
    for b in range(NBUF):
        pltpu.make_async_copy(
            rows.at[b], acc_sh.at[pl.ds(0, CH)], ssems[b]
        ).wait()
    plsc.subcore_barrier()

    nf = RPT // CH
    def fbody(t, carry):
        fds = []
        for b in range(4):
            r0 = s * RPT + (t * 4 + b) * CH
            pltpu.sync_copy(acc_sh.at[pl.ds(r0, CH)], rows.at[b])
            fds.append(pltpu.async_copy(
                rows.at[b], out_hbm.at[pl.ds(c * NPAD + r0, CH)], gsems[b]))
        for d in fds:
            d.wait()
        return carry

    lax.fori_loop(0, nf // 4, fbody, 0)


_BLK = 1024
_GRID = NPAD // _BLK


def _tcmm_body(x_ref, w_ref, h_ref):
    h_ref[...] = jnp.dot(
        x_ref[...], w_ref[...], preferred_element_type=jnp.float32
    )


def _tcmm(xpad, w1):
    return pl.pallas_call(
        _tcmm_body,
        grid=(_GRID,),
        in_specs=[
            pl.BlockSpec((_BLK, D), lambda i: (i, 0)),
            pl.BlockSpec((D, D), lambda i: (0, 0)),
        ],
        out_specs=pl.BlockSpec((_BLK, D), lambda i: (i, 0)),
        out_shape=jax.ShapeDtypeStruct((NPAD, D), jnp.float32),
    )(xpad, w1)


def _tca_body(h_ref, dp_ref, g_ref, dinv_ref):
    deg = jnp.sum(dp_ref[...], axis=0) + 1.0
    dinv = lax.rsqrt(deg)[:, None]
    g_ref[...] = h_ref[...] * dinv
    dinv_ref[...] = dinv


def _tca(h1, degp):
    return pl.pallas_call(
        _tca_body,
        grid=(_GRID,),
        in_specs=[
            pl.BlockSpec((_BLK, D), lambda i: (i, 0)),
            pl.BlockSpec((NC * NS, _BLK), lambda i: (0, i)),
        ],
        out_specs=[
            pl.BlockSpec((_BLK, D), lambda i: (i, 0)),
            pl.BlockSpec((_BLK, 1), lambda i: (i, 0)),
        ],
        out_shape=[
            jax.ShapeDtypeStruct((NPAD, D), jnp.float32),
            jax.ShapeDtypeStruct((NPAD, 1), jnp.float32),
        ],
    )(h1, degp)


def _tcb_body(p0_ref, p1_ref, g1_ref, dinv_ref, b1_ref, w2_ref, g2_ref):
    ssum = p0_ref[...] + p1_ref[...] + g1_ref[...]
    z = jnp.maximum(ssum * dinv_ref[...] + b1_ref[...], 0.0)
    g2_ref[...] = jnp.dot(
        z, w2_ref[...], preferred_element_type=jnp.float32
    ) * dinv_ref[...]


def _tcb(p1, g1, dinv, b1, w2):
    return pl.pallas_call(
        _tcb_body,
        grid=(_GRID,),
        in_specs=[
            pl.BlockSpec((_BLK, D), lambda i: (i, 0)),
            pl.BlockSpec((_BLK, D), lambda i: (i + _GRID, 0)),
            pl.BlockSpec((_BLK, D), lambda i: (i, 0)),
            pl.BlockSpec((_BLK, 1), lambda i: (i, 0)),
            pl.BlockSpec((1, D), lambda i: (0, 0)),
            pl.BlockSpec((D, D), lambda i: (0, 0)),
        ],
        out_specs=pl.BlockSpec((_BLK, D), lambda i: (i, 0)),
        out_shape=jax.ShapeDtypeStruct((NPAD, D), jnp.float32),
    )(p1, p1, g1, dinv, b1, w2)


def _tcc_body(p0_ref, p1_ref, g2_ref, dinv_ref, b2_ref, out_ref):
    ssum = p0_ref[...] + p1_ref[...] + g2_ref[...]
    out_ref[...] = ssum * dinv_ref[...] + b2_ref[...]


def _tcc(p2, g2, dinv, b2):
    return pl.pallas_call(
        _tcc_body,
        grid=(_GRID,),
        in_specs=[
            pl.BlockSpec((_BLK, D), lambda i: (i, 0)),
            pl.BlockSpec((_BLK, D), lambda i: (i + _GRID, 0)),
            pl.BlockSpec((_BLK, D), lambda i: (i, 0)),
            pl.BlockSpec((_BLK, 1), lambda i: (i, 0)),
            pl.BlockSpec((1, D), lambda i: (0, 0)),
        ],
        out_specs=pl.BlockSpec((_BLK, D), lambda i: (i, 0)),
        out_shape=jax.ShapeDtypeStruct((NPAD, D), jnp.float32),
    )(p2, p2, g2, dinv, b2)


def kernel(x, edge_index, W1, b1, W2, b2):
    src = edge_index[0].astype(jnp.int32)
    dst = edge_index[1].astype(jnp.int32)
    xpad = jnp.pad(x, ((0, NPAD - N), (0, 0)))
    b1r = b1.reshape(1, D)
    b2r = b2.reshape(1, D)

    h1 = _tcmm(xpad, W1)
    degp = _deg_kernel(dst).reshape(NC * NS, NPAD)
    g1, dinv = _tca(h1, degp)
    p1 = _agg_kernel(g1, src, dst)
    g2 = _tcb(p1, g1, dinv, b1r, W2)
    p2 = _agg_kernel(g2, src, dst)
    outp = _tcc(p2, g2, dinv, b2r)
    return outp[:N]


# speedup vs baseline: 1.1028x; 1.0431x over previous
import functools

import jax
import jax.numpy as jnp
from jax import lax
from jax.experimental import pallas as pl
from jax.experimental.pallas import tpu as pltpu
from jax.experimental.pallas import tpu_sc as plsc

N = 10000
NPAD = 10240
D = 128
E = 320000
NC = 2
NS = 16
EPT = E // (NC * NS)
CH = 40
NCHUNK = EPT // CH
RPT = NPAD // NS
ZR = 16

_MESH = plsc.VectorSubcoreMesh(
    core_axis_name="c", subcore_axis_name="s", num_cores=NC, num_subcores=NS
)


@functools.partial(
    pl.kernel,
    out_type=jax.ShapeDtypeStruct((NC * NS * NPAD,), jnp.float32),
    mesh=_MESH,
    scratch_types=[
        pltpu.VMEM((EPT,), jnp.int32),
        [pltpu.VMEM((NPAD,), jnp.float32)] * 4,
    ],
)
def _deg_kernel(ei_hbm, out_hbm, idx_v, hists):
    c = lax.axis_index("c")
    s = lax.axis_index("s")
    wid = c * NS + s

    def zfill(i, carry):
        for h in hists:
            h[pl.ds(i * 16, 16)] = jnp.zeros((16,), jnp.float32)
        return carry

    lax.fori_loop(0, NPAD // 16, zfill, 0)
    pltpu.sync_copy(ei_hbm.at[pl.ds(E + wid * EPT, EPT)], idx_v)
    lanes = lax.iota(jnp.int32, 16)

    def body(j, carry):
        dvec = idx_v[pl.ds(j * 16, 16)]
        for k in range(16):
            d = dvec[k]
            row = (d // 16) * 16
            oh = jnp.where(lanes == d % 16, 1.0, 0.0).astype(jnp.float32)
            h = hists[k % 4]
            h[pl.ds(row, 16)] = h[pl.ds(row, 16)] + oh
        return carry

    lax.fori_loop(0, EPT // 16, body, 0)

    def rsum(i, carry):
        sl = pl.ds(i * 16, 16)
        hists[0][sl] = (hists[0][sl] + hists[1][sl]) + (
            hists[2][sl] + hists[3][sl]
        )
        return carry

    lax.fori_loop(0, NPAD // 16, rsum, 0)
    pltpu.sync_copy(hists[0], out_hbm.at[pl.ds(wid * NPAD, NPAD)])


NBUF = 5
NOUT = NCHUNK // NBUF


@functools.partial(
    pl.kernel,
    out_type=jax.ShapeDtypeStruct((NC * NPAD, D), jnp.float32),
    mesh=_MESH,
    scratch_types=[
        pltpu.VMEM((EPT,), jnp.int32),
        pltpu.VMEM((EPT,), jnp.int32),
        pltpu.VMEM((NBUF, CH, D), jnp.float32),
        pltpu.VMEM((ZR, D), jnp.float32),
        pltpu.VMEM_SHARED((NPAD, D), jnp.float32),
        [pltpu.SemaphoreType.DMA] * NBUF,
        [pltpu.SemaphoreType.DMA] * NBUF,
        pltpu.SemaphoreType.DMA,
    ],
)
def _agg_kernel(g_hbm, ei_hbm, out_hbm, srcb, dstb, rows, zero_v,
                acc_sh, gsems, ssems, fsem):
    c = lax.axis_index("c")
    s = lax.axis_index("s")
    wid = c * NS + s
    for r in range(ZR):
        for k in range(D // 16):
            zero_v[r, pl.ds(k * 16, 16)] = jnp.zeros((16,), jnp.float32)

    ebase = wid * EPT
    pd0 = pltpu.async_copy(ei_hbm.at[pl.ds(ebase, EPT)], srcb, gsems[0])
    pd1 = pltpu.async_copy(ei_hbm.at[pl.ds(E + ebase, EPT)], dstb, gsems[1])

    nz = RPT // ZR
    for g in range(2):
        zds = []
        for i in range(nz // 2):
            r0 = s * RPT + (g * (nz // 2) + i) * ZR
            zds.append(pltpu.async_copy(zero_v, acc_sh.at[pl.ds(r0, ZR)],
                                        fsem))
        for d in zds:
            d.wait()
    pd0.wait()
    pd1.wait()
    plsc.subcore_barrier()

    def body(t, carry):
        e0 = t * (NBUF * CH)
        gds = []
        for b in range(NBUF):
            @pl.when(t > 0)
            def _(b=b):
                pltpu.make_async_copy(
                    rows.at[b], acc_sh.at[pl.ds(0, CH)], ssems[b]
                ).wait()

            gds.append(pltpu.async_copy(
                g_hbm.at[srcb.at[pl.ds(e0 + b * CH, CH)]],
                rows.at[b], gsems[b]))
        for b in range(NBUF):
            gds[b].wait()
            pltpu.async_copy(
                rows.at[b],
                acc_sh.at[dstb.at[pl.ds(e0 + b * CH, CH)]],
                ssems[b], add=True)
        return carry

    lax.fori_loop(0, NOUT, body, 0)
    for b in range(NBUF):
        pltpu.make_async_copy(
            rows.at[b], acc_sh.at[pl.ds(0, CH)], ssems[b]
        ).wait()
    plsc.subcore_barrier()

    nf = RPT // CH
    def fbody(t, carry):
        fds = []
        for b in range(4):
            r0 = s * RPT + (t * 4 + b) * CH
            pltpu.sync_copy(acc_sh.at[pl.ds(r0, CH)], rows.at[b])
            fds.append(pltpu.async_copy(
                rows.at[b], out_hbm.at[pl.ds(c * NPAD + r0, CH)], gsems[b]))
        for d in fds:
            d.wait()
        return carry

    lax.fori_loop(0, nf // 4, fbody, 0)


_BLK = 1024
_GRID = NPAD // _BLK


def _tcmm_body(x_ref, w_ref, h_ref):
    h_ref[...] = jnp.dot(
        x_ref[...], w_ref[...], preferred_element_type=jnp.float32
    )


def _tcmm(x, w1):
    return pl.pallas_call(
        _tcmm_body,
        grid=(_GRID,),
        in_specs=[
            pl.BlockSpec((_BLK, D), lambda i: (i, 0)),
            pl.BlockSpec((D, D), lambda i: (0, 0)),
        ],
        out_specs=pl.BlockSpec((_BLK, D), lambda i: (i, 0)),
        out_shape=jax.ShapeDtypeStruct((NPAD, D), jnp.float32),
    )(x, w1)


def _tca_body(h_ref, dp_ref, g_ref, dinv_ref):
    deg = jnp.sum(dp_ref[...], axis=0) + 1.0
    dinv = lax.rsqrt(deg)[:, None]
    g_ref[...] = h_ref[...] * dinv
    dinv_ref[...] = dinv


def _tca(h1, degp):
    return pl.pallas_call(
        _tca_body,
        grid=(_GRID,),
        in_specs=[
            pl.BlockSpec((_BLK, D), lambda i: (i, 0)),
            pl.BlockSpec((NC * NS, _BLK), lambda i: (0, i)),
        ],
        out_specs=[
            pl.BlockSpec((_BLK, D), lambda i: (i, 0)),
            pl.BlockSpec((_BLK, 1), lambda i: (i, 0)),
        ],
        out_shape=[
            jax.ShapeDtypeStruct((NPAD, D), jnp.float32),
            jax.ShapeDtypeStruct((NPAD, 1), jnp.float32),
        ],
    )(h1, degp)


def _tcb_body(p0_ref, p1_ref, g1_ref, dinv_ref, b1_ref, w2_ref, g2_ref):
    ssum = p0_ref[...] + p1_ref[...] + g1_ref[...]
    z = jnp.maximum(ssum * dinv_ref[...] + b1_ref[...], 0.0)
    g2_ref[...] = jnp.dot(
        z, w2_ref[...], preferred_element_type=jnp.float32
    ) * dinv_ref[...]


def _tcb(p1, g1, dinv, b1, w2):
    return pl.pallas_call(
        _tcb_body,
        grid=(_GRID,),
        in_specs=[
            pl.BlockSpec((_BLK, D), lambda i: (i, 0)),
            pl.BlockSpec((_BLK, D), lambda i: (i + _GRID, 0)),
            pl.BlockSpec((_BLK, D), lambda i: (i, 0)),
            pl.BlockSpec((_BLK, 1), lambda i: (i, 0)),
            pl.BlockSpec((1, D), lambda i: (0, 0)),
            pl.BlockSpec((D, D), lambda i: (0, 0)),
        ],
        out_specs=pl.BlockSpec((_BLK, D), lambda i: (i, 0)),
        out_shape=jax.ShapeDtypeStruct((NPAD, D), jnp.float32),
    )(p1, p1, g1, dinv, b1, w2)


def _tcc_body(p0_ref, p1_ref, g2_ref, dinv_ref, b2_ref, out_ref):
    ssum = p0_ref[...] + p1_ref[...] + g2_ref[...]
    out_ref[...] = ssum * dinv_ref[...] + b2_ref[...]


def _tcc(p2, g2, dinv, b2):
    return pl.pallas_call(
        _tcc_body,
        grid=(_GRID,),
        in_specs=[
            pl.BlockSpec((_BLK, D), lambda i: (i, 0)),
            pl.BlockSpec((_BLK, D), lambda i: (i + _GRID, 0)),
            pl.BlockSpec((_BLK, D), lambda i: (i, 0)),
            pl.BlockSpec((_BLK, 1), lambda i: (i, 0)),
            pl.BlockSpec((1, D), lambda i: (0, 0)),
        ],
        out_specs=pl.BlockSpec((_BLK, D), lambda i: (i, 0)),
        out_shape=jax.ShapeDtypeStruct((N, D), jnp.float32),
    )(p2, p2, g2, dinv, b2)


def kernel(x, edge_index, W1, b1, W2, b2):
    ei = edge_index.astype(jnp.int32).reshape(2 * E)
    b1r = b1.reshape(1, D)
    b2r = b2.reshape(1, D)

    h1 = _tcmm(x, W1)
    degp = _deg_kernel(ei).reshape(NC * NS, NPAD)
    g1, dinv = _tca(h1, degp)
    p1 = _agg_kernel(g1, ei)
    g2 = _tcb(p1, g1, dinv, b1r, W2)
    p2 = _agg_kernel(g2, ei)
    return _tcc(p2, g2, dinv, b2r)
